# baseline (speedup 1.0000x reference)
import functools

import jax
import jax.numpy as jnp
from jax import lax
from jax.experimental import pallas as pl
from jax.experimental.pallas import tpu as pltpu
from jax.experimental.pallas import tpu_sc as plsc

N = 10000
E = 320000
D = 128
D_FF = 2 * D

NC = 2
NS = 16
NW = NC * NS
EPW = E // NW
CH = 80
NCHUNK = EPW // CH
RPT = 640
NPAD = NS * RPT
ZCOPIES = RPT // CH

_MESH = plsc.VectorSubcoreMesh(core_axis_name="c", subcore_axis_name="s")


def _deg_body(col_hbm, out_hbm, idx_v, ones_v, zero_v, deg_sh, isem, asem):
    cid = lax.axis_index("c")
    sid = lax.axis_index("s")
    wid = sid * NC + cid

    pltpu.async_copy(col_hbm.at[wid], idx_v, isem)

    def fill_ones(i, _):
        ones_v[pl.ds(i * 16, 16)] = jnp.ones((16,), jnp.float32)
        return 0

    lax.fori_loop(0, CH // 16, fill_ones, 0)

    def fill_zero(i, _):
        zero_v[pl.ds(i * 16, 16)] = jnp.zeros((16,), jnp.float32)
        return 0

    lax.fori_loop(0, RPT // 16, fill_zero, 0)

    pltpu.sync_copy(zero_v, deg_sh.at[pl.ds(sid * RPT, RPT)])
    plsc.subcore_barrier()
    pltpu.make_async_copy(col_hbm.at[wid], idx_v, isem).wait()

    LAG = 8

    def fire(j, _):
        pltpu.async_copy(ones_v, deg_sh.at[idx_v.at[j]], asem, add=True)
        return 0

    def fire_drain(j, _):
        pltpu.async_copy(ones_v, deg_sh.at[idx_v.at[j]], asem, add=True)
        pltpu.make_async_copy(ones_v, deg_sh.at[idx_v.at[0]], asem).wait()
        return 0

    def drain(j, _):
        pltpu.make_async_copy(ones_v, deg_sh.at[idx_v.at[0]], asem).wait()
        return 0

    lax.fori_loop(0, LAG, fire, 0)
    lax.fori_loop(LAG, NCHUNK, fire_drain, 0)
    lax.fori_loop(0, LAG, drain, 0)
    plsc.subcore_barrier()

    pltpu.sync_copy(deg_sh.at[pl.ds(sid * RPT, RPT)],
                    out_hbm.at[cid, pl.ds(sid * RPT, RPT)])


_deg_call = pl.kernel(
    _deg_body,
    out_type=jax.ShapeDtypeStruct((NC, NPAD), jnp.float32),
    mesh=_MESH,
    scratch_types=[
        pltpu.VMEM((NCHUNK, CH), jnp.int32),
        pltpu.VMEM((CH,), jnp.float32),
        pltpu.VMEM((RPT,), jnp.float32),
        pltpu.VMEM_SHARED((NPAD,), jnp.float32),
        pltpu.SemaphoreType.DMA,
        pltpu.SemaphoreType.DMA,
    ],
)


def _z_body(y_hbm, pidx_hbm, out_hbm, pidx_v, rbuf0, cbuf0, rbuf1, cbuf1,
            gbuf0, gbuf1, z_sh, gsem0, gsem1, ssem0, ssem1, zsem, isem,
            gsem0b, gsem1b, gsem0c, gsem0d, gsem1c, gsem1d):
    QOFF = (0, 24, 48, 64)
    QSZ = (24, 24, 16, 16)
    g0sems = (gsem0, gsem0b, gsem0c, gsem0d)
    g1sems = (gsem1, gsem1b, gsem1c, gsem1d)
    cid = lax.axis_index("c")
    sid = lax.axis_index("s")
    wid = sid * NC + cid

    pltpu.async_copy(pidx_hbm.at[wid], pidx_v, isem)

    def fill_zero(r, _):
        for c in range(D // 16):
            gbuf1[r, pl.ds(c * 16, 16)] = jnp.zeros((16,), jnp.float32)
        return 0

    lax.fori_loop(0, CH, fill_zero, 0)

    for k in range(ZCOPIES):
        pltpu.async_copy(gbuf1, z_sh.at[pl.ds(sid * RPT + k * CH, CH)], zsem)

    def unpack(j, rbuf, cbuf):
        def body(t, _):
            v = pidx_v[j, pl.ds(t * 16, 16)]
            rbuf[pl.ds(t * 16, 16)] = jnp.bitwise_and(v, 0xFFFF)
            cbuf[pl.ds(t * 16, 16)] = lax.shift_right_logical(v, 16)
            return 0

        lax.fori_loop(0, CH // 16, body, 0)

    def start_g0():
        for q in range(4):
            pltpu.async_copy(y_hbm.at[rbuf0.at[pl.ds(QOFF[q], QSZ[q])]],
                             gbuf0.at[pl.ds(QOFF[q], QSZ[q])], g0sems[q])

    def start_g1():
        for q in range(4):
            pltpu.async_copy(y_hbm.at[rbuf1.at[pl.ds(QOFF[q], QSZ[q])]],
                             gbuf1.at[pl.ds(QOFF[q], QSZ[q])], g1sems[q])

    def wait_g0():
        for q in range(4):
            pltpu.make_async_copy(
                y_hbm.at[rbuf0.at[pl.ds(QOFF[q], QSZ[q])]],
                gbuf0.at[pl.ds(QOFF[q], QSZ[q])], g0sems[q]).wait()

    def wait_g1():
        for q in range(4):
            pltpu.make_async_copy(
                y_hbm.at[rbuf1.at[pl.ds(QOFF[q], QSZ[q])]],
                gbuf1.at[pl.ds(QOFF[q], QSZ[q])], g1sems[q]).wait()

    def wait_s0():
        pltpu.make_async_copy(gbuf0, z_sh.at[cbuf0], ssem0).wait()

    def wait_s1():
        pltpu.make_async_copy(gbuf1, z_sh.at[cbuf1], ssem1).wait()

    pltpu.make_async_copy(pidx_hbm.at[wid], pidx_v, isem).wait()
    unpack(0, rbuf0, cbuf0)
    start_g0()
    for k in range(ZCOPIES):
        pltpu.make_async_copy(
            gbuf1, z_sh.at[pl.ds(sid * RPT + k * CH, CH)], zsem).wait()
    plsc.subcore_barrier()

    unpack(1, rbuf1, cbuf1)
    start_g1()
    wait_g0()
    pltpu.async_copy(gbuf0, z_sh.at[cbuf0], ssem0, add=True)
    wait_g1()
    pltpu.async_copy(gbuf1, z_sh.at[cbuf1], ssem1, add=True)
    wait_s0()
    unpack(2, rbuf0, cbuf0)
    start_g0()

    def pair(k, _):
        a = 2 * k
        wait_g0()
        pltpu.async_copy(gbuf0, z_sh.at[cbuf0], ssem0, add=True)
        wait_s1()
        unpack(a + 1, rbuf1, cbuf1)
        start_g1()
        wait_g1()
        pltpu.async_copy(gbuf1, z_sh.at[cbuf1], ssem1, add=True)
        wait_s0()
        unpack(a + 2, rbuf0, cbuf0)
        start_g0()
        return 0

    lax.fori_loop(1, (NCHUNK - 1) // 2, pair, 0)

    wait_g0()
    pltpu.async_copy(gbuf0, z_sh.at[cbuf0], ssem0, add=True)
    wait_s1()
    wait_s0()
    plsc.subcore_barrier()

    pltpu.sync_copy(z_sh.at[pl.ds(sid * RPT, RPT)],
                    out_hbm.at[cid, pl.ds(sid * RPT, RPT)])


_z_call = pl.kernel(
    _z_body,
    out_type=jax.ShapeDtypeStruct((NC, NPAD, D), jnp.float32),
    mesh=_MESH,
    scratch_types=[
        pltpu.VMEM((NCHUNK, CH), jnp.int32),
        pltpu.VMEM((CH,), jnp.int32),
        pltpu.VMEM((CH,), jnp.int32),
        pltpu.VMEM((CH,), jnp.int32),
        pltpu.VMEM((CH,), jnp.int32),
        pltpu.VMEM((CH, D), jnp.float32),
        pltpu.VMEM((CH, D), jnp.float32),
        pltpu.VMEM_SHARED((NPAD, D), jnp.float32),
        pltpu.SemaphoreType.DMA,
        pltpu.SemaphoreType.DMA,
        pltpu.SemaphoreType.DMA,
        pltpu.SemaphoreType.DMA,
        pltpu.SemaphoreType.DMA,
        pltpu.SemaphoreType.DMA,
        pltpu.SemaphoreType.DMA,
        pltpu.SemaphoreType.DMA,
        pltpu.SemaphoreType.DMA,
        pltpu.SemaphoreType.DMA,
        pltpu.SemaphoreType.DMA,
        pltpu.SemaphoreType.DMA,
    ],
)


def _xlin_body(x_ref, w_ref, o_ref):
    o_ref[...] = jnp.dot(x_ref[...], w_ref[...],
                         preferred_element_type=jnp.float32)


def _xlin_call(x, w):
    return pl.pallas_call(
        _xlin_body,
        out_shape=jax.ShapeDtypeStruct((N, D), jnp.float32),
    )(x, w)


def _y_body(xl_ref, deg_ref, y_ref):
    deg = deg_ref[0, :N] + deg_ref[1, :N] + 1.0
    dis = lax.rsqrt(deg)
    y_ref[...] = xl_ref[...] * dis[:, None]


def _y_call(xl, deg_p):
    return pl.pallas_call(
        _y_body,
        out_shape=jax.ShapeDtypeStruct((N, D), jnp.float32),
    )(xl, deg_p)


def _final_body(x_ref, y_ref, deg_ref, z_ref, b_gcn_ref, g1_ref, b1_ref,
                wf1_ref, bf1_ref, wf2_ref, bf2_ref, g2_ref, b2_ref, out_ref):
    deg = deg_ref[0, :N] + deg_ref[1, :N] + 1.0
    dis = lax.rsqrt(deg)
    y = y_ref[...]
    z = z_ref[0, :N, :] + z_ref[1, :N, :]
    gcn = (z + y) * dis[:, None] + b_gcn_ref[...]
    h = x_ref[...] + gcn

    mu1 = jnp.mean(h, axis=0)
    hc = h - mu1
    var1 = jnp.mean(hc * hc, axis=0)
    h1 = hc * lax.rsqrt(var1 + 1e-5) * g1_ref[...] + b1_ref[...]

    mid = jnp.maximum(
        jnp.dot(h1, wf1_ref[...], preferred_element_type=jnp.float32)
        + bf1_ref[...], 0.0)
    ff = jnp.dot(mid, wf2_ref[...],
                 preferred_element_type=jnp.float32) + bf2_ref[...]
    h2 = h1 + ff

    mu2 = jnp.mean(h2, axis=0)
    hc2 = h2 - mu2
    var2 = jnp.mean(hc2 * hc2, axis=0)
    out_ref[...] = hc2 * lax.rsqrt(var2 + 1e-5) * g2_ref[...] + b2_ref[...]


def _final_call(x, y, deg_p, z_p, b_gcn, g1, b1, wf1, bf1, wf2, bf2, g2, b2):
    return pl.pallas_call(
        _final_body,
        out_shape=jax.ShapeDtypeStruct((N, D), jnp.float32),
    )(x, y, deg_p, z_p, b_gcn, g1, b1, wf1, bf1, wf2, bf2, g2, b2)


def kernel(x, edge_index, W_gcn, b_gcn, bn1_gamma, bn1_beta, W_ff1, b_ff1,
           W_ff2, b_ff2, bn2_gamma, bn2_beta):
    ei = edge_index.astype(jnp.int32)
    col = ei[1].reshape(NW, NCHUNK, CH)
    pidx = jnp.bitwise_or(ei[0], jnp.left_shift(ei[1], 16))
    pidx = pidx.reshape(NW, NCHUNK, CH)
    xl = _xlin_call(x, W_gcn)
    deg_p = _deg_call(col)
    y = _y_call(xl, deg_p)
    z_p = _z_call(y, pidx)
    return _final_call(x, y, deg_p, z_p, b_gcn, bn1_gamma, bn1_beta,
                       W_ff1, b_ff1, W_ff2, b_ff2, bn2_gamma, bn2_beta)


# speedup vs baseline: 1.0259x; 1.0259x over previous
import functools

import jax
import jax.numpy as jnp
from jax import lax
from jax.experimental import pallas as pl
from jax.experimental.pallas import tpu as pltpu
from jax.experimental.pallas import tpu_sc as plsc

N = 10000
E = 320000
D = 128
D_FF = 2 * D

NC = 2
NS = 16
NW = NC * NS
EPW = E // NW
CH = 80
NCHUNK = EPW // CH
RPT = 640
NPAD = NS * RPT
ZCOPIES = RPT // CH

_MESH = plsc.VectorSubcoreMesh(core_axis_name="c", subcore_axis_name="s")


def _deg_body(col_hbm, out_hbm, idx_v, ones_v, zero_v, deg_sh, isem, asem):
    cid = lax.axis_index("c")
    sid = lax.axis_index("s")
    wid = sid * NC + cid

    pltpu.async_copy(col_hbm.at[wid], idx_v, isem)

    def fill_ones(i, _):
        ones_v[pl.ds(i * 16, 16)] = jnp.ones((16,), jnp.float32)
        return 0

    lax.fori_loop(0, CH // 16, fill_ones, 0)

    def fill_zero(i, _):
        zero_v[pl.ds(i * 16, 16)] = jnp.zeros((16,), jnp.float32)
        return 0

    lax.fori_loop(0, RPT // 16, fill_zero, 0)

    pltpu.sync_copy(zero_v, deg_sh.at[pl.ds(sid * RPT, RPT)])
    plsc.subcore_barrier()
    pltpu.make_async_copy(col_hbm.at[wid], idx_v, isem).wait()

    LAG = 8

    def fire(j, _):
        pltpu.async_copy(ones_v, deg_sh.at[idx_v.at[j]], asem, add=True)
        return 0

    def fire_drain(j, _):
        pltpu.async_copy(ones_v, deg_sh.at[idx_v.at[j]], asem, add=True)
        pltpu.make_async_copy(ones_v, deg_sh.at[idx_v.at[0]], asem).wait()
        return 0

    def drain(j, _):
        pltpu.make_async_copy(ones_v, deg_sh.at[idx_v.at[0]], asem).wait()
        return 0

    lax.fori_loop(0, LAG, fire, 0)
    lax.fori_loop(LAG, NCHUNK, fire_drain, 0)
    lax.fori_loop(0, LAG, drain, 0)
    plsc.subcore_barrier()

    pltpu.sync_copy(deg_sh.at[pl.ds(sid * RPT, RPT)],
                    out_hbm.at[cid, pl.ds(sid * RPT, RPT)])


_deg_call = pl.kernel(
    _deg_body,
    out_type=jax.ShapeDtypeStruct((NC, NPAD), jnp.float32),
    mesh=_MESH,
    scratch_types=[
        pltpu.VMEM((NCHUNK, CH), jnp.int32),
        pltpu.VMEM((CH,), jnp.float32),
        pltpu.VMEM((RPT,), jnp.float32),
        pltpu.VMEM_SHARED((NPAD,), jnp.float32),
        pltpu.SemaphoreType.DMA,
        pltpu.SemaphoreType.DMA,
    ],
)


def _z_body(y_hbm, pidx_hbm, out_hbm, pidx_v, rbuf0, cbuf0, rbuf1, cbuf1,
            gbuf0, gbuf1, z_sh, gsem0, gsem1, ssem0, ssem1, zsem, isem,
            gsem0b, gsem1b):
    HH = CH // 2
    cid = lax.axis_index("c")
    sid = lax.axis_index("s")
    wid = sid * NC + cid

    pltpu.async_copy(pidx_hbm.at[wid], pidx_v, isem)

    def fill_zero(r, _):
        for c in range(D // 16):
            gbuf1[r, pl.ds(c * 16, 16)] = jnp.zeros((16,), jnp.float32)
        return 0

    lax.fori_loop(0, CH, fill_zero, 0)

    for k in range(ZCOPIES):
        pltpu.async_copy(gbuf1, z_sh.at[pl.ds(sid * RPT + k * CH, CH)], zsem)

    def unpack(j, rbuf, cbuf):
        def body(t, _):
            v = pidx_v[j, pl.ds(t * 16, 16)]
            rbuf[pl.ds(t * 16, 16)] = jnp.bitwise_and(v, 0xFFFF)
            cbuf[pl.ds(t * 16, 16)] = lax.shift_right_logical(v, 16)
            return 0

        lax.fori_loop(0, CH // 16, body, 0)

    def start_g0():
        pltpu.async_copy(y_hbm.at[rbuf0.at[pl.ds(0, HH)]],
                         gbuf0.at[pl.ds(0, HH)], gsem0)
        pltpu.async_copy(y_hbm.at[rbuf0.at[pl.ds(HH, HH)]],
                         gbuf0.at[pl.ds(HH, HH)], gsem0b)

    def start_g1():
        pltpu.async_copy(y_hbm.at[rbuf1.at[pl.ds(0, HH)]],
                         gbuf1.at[pl.ds(0, HH)], gsem1)
        pltpu.async_copy(y_hbm.at[rbuf1.at[pl.ds(HH, HH)]],
                         gbuf1.at[pl.ds(HH, HH)], gsem1b)

    def wait_g0():
        pltpu.make_async_copy(y_hbm.at[rbuf0.at[pl.ds(0, HH)]],
                              gbuf0.at[pl.ds(0, HH)], gsem0).wait()
        pltpu.make_async_copy(y_hbm.at[rbuf0.at[pl.ds(HH, HH)]],
                              gbuf0.at[pl.ds(HH, HH)], gsem0b).wait()

    def wait_g1():
        pltpu.make_async_copy(y_hbm.at[rbuf1.at[pl.ds(0, HH)]],
                              gbuf1.at[pl.ds(0, HH)], gsem1).wait()
        pltpu.make_async_copy(y_hbm.at[rbuf1.at[pl.ds(HH, HH)]],
                              gbuf1.at[pl.ds(HH, HH)], gsem1b).wait()

    def wait_s0():
        pltpu.make_async_copy(gbuf0, z_sh.at[cbuf0], ssem0).wait()

    def wait_s1():
        pltpu.make_async_copy(gbuf1, z_sh.at[cbuf1], ssem1).wait()

    pltpu.make_async_copy(pidx_hbm.at[wid], pidx_v, isem).wait()
    unpack(0, rbuf0, cbuf0)
    start_g0()
    for k in range(ZCOPIES):
        pltpu.make_async_copy(
            gbuf1, z_sh.at[pl.ds(sid * RPT + k * CH, CH)], zsem).wait()
    plsc.subcore_barrier()

    unpack(1, rbuf1, cbuf1)
    start_g1()
    wait_g0()
    pltpu.async_copy(gbuf0, z_sh.at[cbuf0], ssem0, add=True)
    wait_g1()
    pltpu.async_copy(gbuf1, z_sh.at[cbuf1], ssem1, add=True)
    wait_s0()
    unpack(2, rbuf0, cbuf0)
    start_g0()

    def pair(k, _):
        a = 2 * k
        wait_g0()
        pltpu.async_copy(gbuf0, z_sh.at[cbuf0], ssem0, add=True)
        wait_s1()
        unpack(a + 1, rbuf1, cbuf1)
        start_g1()
        wait_g1()
        pltpu.async_copy(gbuf1, z_sh.at[cbuf1], ssem1, add=True)
        wait_s0()
        unpack(a + 2, rbuf0, cbuf0)
        start_g0()
        return 0

    lax.fori_loop(1, (NCHUNK - 1) // 2, pair, 0)

    wait_g0()
    pltpu.async_copy(gbuf0, z_sh.at[cbuf0], ssem0, add=True)
    wait_s1()
    wait_s0()
    plsc.subcore_barrier()

    pltpu.sync_copy(z_sh.at[pl.ds(sid * RPT, RPT)],
                    out_hbm.at[cid, pl.ds(sid * RPT, RPT)])


_z_call = pl.kernel(
    _z_body,
    out_type=jax.ShapeDtypeStruct((NC, NPAD, D), jnp.float32),
    mesh=_MESH,
    scratch_types=[
        pltpu.VMEM((NCHUNK, CH), jnp.int32),
        pltpu.VMEM((CH,), jnp.int32),
        pltpu.VMEM((CH,), jnp.int32),
        pltpu.VMEM((CH,), jnp.int32),
        pltpu.VMEM((CH,), jnp.int32),
        pltpu.VMEM((CH, D), jnp.float32),
        pltpu.VMEM((CH, D), jnp.float32),
        pltpu.VMEM_SHARED((NPAD, D), jnp.float32),
        pltpu.SemaphoreType.DMA,
        pltpu.SemaphoreType.DMA,
        pltpu.SemaphoreType.DMA,
        pltpu.SemaphoreType.DMA,
        pltpu.SemaphoreType.DMA,
        pltpu.SemaphoreType.DMA,
        pltpu.SemaphoreType.DMA,
        pltpu.SemaphoreType.DMA,
    ],
)


def _xliny_body(x_ref, w_ref, deg_ref, y_ref):
    deg = deg_ref[0, :N] + deg_ref[1, :N] + 1.0
    dis = lax.rsqrt(deg)
    xl = jnp.dot(x_ref[...], w_ref[...], preferred_element_type=jnp.float32)
    y_ref[...] = xl * dis[:, None]


def _xliny_call(x, w, deg_p):
    return pl.pallas_call(
        _xliny_body,
        out_shape=jax.ShapeDtypeStruct((N, D), jnp.float32),
    )(x, w, deg_p)


def _final_body(x_ref, y_ref, deg_ref, z_ref, b_gcn_ref, g1_ref, b1_ref,
                wf1_ref, bf1_ref, wf2_ref, bf2_ref, g2_ref, b2_ref, out_ref):
    deg = deg_ref[0, :N] + deg_ref[1, :N] + 1.0
    dis = lax.rsqrt(deg)
    y = y_ref[...]
    z = z_ref[0, :N, :] + z_ref[1, :N, :]
    gcn = (z + y) * dis[:, None] + b_gcn_ref[...]
    h = x_ref[...] + gcn

    mu1 = jnp.mean(h, axis=0)
    hc = h - mu1
    var1 = jnp.mean(hc * hc, axis=0)
    h1 = hc * lax.rsqrt(var1 + 1e-5) * g1_ref[...] + b1_ref[...]

    mid = jnp.maximum(
        jnp.dot(h1, wf1_ref[...], preferred_element_type=jnp.float32)
        + bf1_ref[...], 0.0)
    ff = jnp.dot(mid, wf2_ref[...],
                 preferred_element_type=jnp.float32) + bf2_ref[...]
    h2 = h1 + ff

    mu2 = jnp.mean(h2, axis=0)
    hc2 = h2 - mu2
    var2 = jnp.mean(hc2 * hc2, axis=0)
    out_ref[...] = hc2 * lax.rsqrt(var2 + 1e-5) * g2_ref[...] + b2_ref[...]


def _final_call(x, y, deg_p, z_p, b_gcn, g1, b1, wf1, bf1, wf2, bf2, g2, b2):
    return pl.pallas_call(
        _final_body,
        out_shape=jax.ShapeDtypeStruct((N, D), jnp.float32),
    )(x, y, deg_p, z_p, b_gcn, g1, b1, wf1, bf1, wf2, bf2, g2, b2)


def kernel(x, edge_index, W_gcn, b_gcn, bn1_gamma, bn1_beta, W_ff1, b_ff1,
           W_ff2, b_ff2, bn2_gamma, bn2_beta):
    ei = edge_index.astype(jnp.int32)
    col = ei[1].reshape(NW, NCHUNK, CH)
    pidx = jnp.bitwise_or(ei[0], jnp.left_shift(ei[1], 16))
    pidx = pidx.reshape(NW, NCHUNK, CH)
    deg_p = _deg_call(col)
    y = _xliny_call(x, W_gcn, deg_p)
    z_p = _z_call(y, pidx)
    return _final_call(x, y, deg_p, z_p, b_gcn, bn1_gamma, bn1_beta,
                       W_ff1, b_ff1, W_ff2, b_ff2, bn2_gamma, bn2_beta)
